# Initial kernel scaffold; baseline (speedup 1.0000x reference)
import jax
import jax.numpy as jnp
from jax.experimental import pallas as pl


def kernel(h, edge_index, ntype, etype, Wk, Wq, Wv, Wa, rel_pri, rel_att, rel_msg, skip, gamma, beta):
    raise NotImplementedError("write your pallas kernel here")


# speedup vs baseline: 30.8384x; 30.8384x over previous
import numpy as np
import jax
import jax.numpy as jnp
from jax import lax
from jax.experimental import pallas as pl
from jax.experimental.pallas import tpu as pltpu
from jax.experimental.pallas import tpu_sc as plsc

_N, _E, _D, _H, _HS, _NT, _ET = 10000, 320000, 128, 8, 16, 3, 5

_NC, _NS, _L = 2, 16, 16
_C = 40
_EPW = _E // (_NC * _NS)
_NCH = _EPW // _C
_NPAD = 10240
_RPT = _NPAD // _NS
_DN = _NPAD // 8
_DRPT = _DN // _NS
_BN = 1000

_J = np.array([(p % 8) * 16 + p // 8 for p in range(_D)], dtype=np.int32)
_M_NP = np.zeros((_L, _D), dtype=np.float32)
_M_NP[np.arange(_D) % _H, np.arange(_D)] = 1.0


def _dense_in_body(h_ref, nt_ref, wk_ref, wq_ref, wv_ref, ra_ref, rm_ref,
                   q_ref, kr_ref, vr_ref):
    x = h_ref[...]
    nt = nt_ref[...]

    def typed(w_ref):
        out = jnp.zeros((x.shape[0], _D), jnp.float32)
        for t in range(_NT):
            out = out + jnp.where(
                nt == t, jnp.dot(x, w_ref[t], preferred_element_type=jnp.float32), 0.0)
        return out

    k = typed(wk_ref)
    v = typed(wv_ref)
    q_ref[...] = typed(wq_ref)
    kr_ref[...] = jnp.dot(k, ra_ref[...], preferred_element_type=jnp.float32)
    vr_ref[...] = jnp.dot(v, rm_ref[...], preferred_element_type=jnp.float32)


def _dense_in(h, nt2, Wk, WqP, Wv, RattPs, RmsgP):
    full3 = pl.BlockSpec((_NT, _D, _D), lambda i: (0, 0, 0))
    rspec = pl.BlockSpec((_D, _ET * _D), lambda i: (0, 0))
    return pl.pallas_call(
        _dense_in_body,
        grid=(_N // _BN,),
        in_specs=[pl.BlockSpec((_BN, _D), lambda i: (i, 0)),
                  pl.BlockSpec((_BN, 1), lambda i: (i, 0)),
                  full3, full3, full3, rspec, rspec],
        out_specs=[pl.BlockSpec((_BN, _D), lambda i: (i, 0)),
                   pl.BlockSpec((_BN, _ET * _D), lambda i: (i, 0)),
                   pl.BlockSpec((_BN, _ET * _D), lambda i: (i, 0))],
        out_shape=[jax.ShapeDtypeStruct((_N, _D), jnp.float32),
                   jax.ShapeDtypeStruct((_N, _ET * _D), jnp.float32),
                   jax.ShapeDtypeStruct((_N, _ET * _D), jnp.float32)],
    )(h, nt2, Wk, WqP, Wv, RattPs, RmsgP)


def _edge_body(krelT, vrelT, qP, src, dst, et, accout, denout,
               acc_sh, den_sh, sbuf, dbuf, dpad, ebuf, ibuf, nbuf,
               kbuf, qbuf, vbuf, wbuf, debuf, sem0, sem1, sem2):
    cid = lax.axis_index("c")
    sid = lax.axis_index("s")
    iot = lax.iota(jnp.int32, _L)
    z16 = jnp.zeros((_L,), jnp.float32)

    def zrow(r, carry):
        for v2 in range(_D // _L):
            wbuf[r, pl.ds(v2 * _L, _L)] = z16
            debuf[r, pl.ds(v2 * _L, _L)] = z16
        return carry
    lax.fori_loop(0, _C, zrow, 0)
    for b in range(_RPT // _C):
        pltpu.sync_copy(wbuf, acc_sh.at[pl.ds(sid * _RPT + b * _C, _C)])
    for b in range(_DRPT // _C):
        pltpu.sync_copy(wbuf, den_sh.at[pl.ds(sid * _DRPT + b * _C, _C)])
    plsc.subcore_barrier()

    base = (cid * _NS + sid) * _EPW
    perm = iot ^ 8
    gdn = lax.GatherDimensionNumbers(
        offset_dims=(), collapsed_slice_dims=(0,), start_index_map=(0,))
    goffs = []
    o = 0
    while o + _L < _C:
        goffs.append(o)
        o += _L
    goffs.append(_C - _L)

    def chunk(j, carry):
        off = base + j * _C
        pltpu.sync_copy(src.at[pl.ds(off, _C)], sbuf)
        pltpu.sync_copy(dst.at[pl.ds(off, _C)], dbuf)
        pltpu.sync_copy(dst.at[pl.ds(off, _C)], dpad.at[pl.ds(0, _C)])
        pltpu.sync_copy(et.at[pl.ds(off, _C)], ebuf)
        for go in goffs:
            sl = pl.ds(go, _L)
            ibuf[sl] = sbuf[sl] * _ET + ebuf[sl]
            nbuf[sl] = lax.shift_right_logical(dbuf[sl], 3)
        cp0 = pltpu.async_copy(krelT.at[ibuf], kbuf, sem0)
        cp1 = pltpu.async_copy(qP.at[dbuf], qbuf, sem1)
        cp2 = pltpu.async_copy(vrelT.at[ibuf], vbuf, sem2)
        cp0.wait()
        cp1.wait()
        cp2.wait()

        def edge(c, ecarry):
            s16 = kbuf[c, pl.ds(0, _L)] * qbuf[c, pl.ds(0, _L)]
            for v2 in range(1, _D // _L):
                sl = pl.ds(v2 * _L, _L)
                s16 = s16 + kbuf[c, sl] * qbuf[c, sl]
            a16 = s16 + lax.gather(
                s16, perm[:, None], dimension_numbers=gdn, slice_sizes=(1,),
                mode=lax.GatherScatterMode.PROMISE_IN_BOUNDS)
            ex = jnp.exp(a16)
            for v2 in range(_D // _L):
                sl = pl.ds(v2 * _L, _L)
                wbuf[c, sl] = vbuf[c, sl] * ex
            slot = (dpad[pl.ds(c, _L)][0] & 7) * _L
            debuf[c, pl.ds(slot, _L)] = ex
            return ecarry
        lax.fori_loop(0, _C, edge, 0)

        pltpu.sync_copy(wbuf, acc_sh.at[dbuf], add=True)
        pltpu.sync_copy(debuf, den_sh.at[nbuf], add=True)

        def dz(r, ecarry):
            for v2 in range(_D // _L):
                debuf[r, pl.ds(v2 * _L, _L)] = z16
            return ecarry
        lax.fori_loop(0, _C, dz, 0)
        return carry
    lax.fori_loop(0, _NCH, chunk, 0)

    plsc.subcore_barrier()
    pltpu.sync_copy(acc_sh.at[pl.ds(sid * _RPT, _RPT)],
                    accout.at[cid, pl.ds(sid * _RPT, _RPT)])
    pltpu.sync_copy(den_sh.at[pl.ds(sid * _DRPT, _DRPT)],
                    denout.at[cid, pl.ds(sid * _DRPT, _DRPT)])


def _edge_stage(krelT, vrelT, qP, src, dst, et):
    kern = pl.kernel(
        _edge_body,
        out_type=(jax.ShapeDtypeStruct((_NC, _NPAD, _D), jnp.float32),
                  jax.ShapeDtypeStruct((_NC, _DN, _D), jnp.float32)),
        mesh=plsc.VectorSubcoreMesh(core_axis_name="c", subcore_axis_name="s",
                                    num_cores=_NC, num_subcores=_NS),
        scratch_types=[
            pltpu.VMEM_SHARED((_NPAD, _D), jnp.float32),
            pltpu.VMEM_SHARED((_DN, _D), jnp.float32),
            pltpu.VMEM((_C,), jnp.int32),
            pltpu.VMEM((_C,), jnp.int32),
            pltpu.VMEM((_C + _L,), jnp.int32),
            pltpu.VMEM((_C,), jnp.int32),
            pltpu.VMEM((_C,), jnp.int32),
            pltpu.VMEM((_C,), jnp.int32),
            pltpu.VMEM((_C, _D), jnp.float32),
            pltpu.VMEM((_C, _D), jnp.float32),
            pltpu.VMEM((_C, _D), jnp.float32),
            pltpu.VMEM((_C, _D), jnp.float32),
            pltpu.VMEM((_C, _D), jnp.float32),
            pltpu.SemaphoreType.DMA,
            pltpu.SemaphoreType.DMA,
            pltpu.SemaphoreType.DMA,
        ],
    )
    return kern(krelT, vrelT, qP, src, dst, et)


def _dense_out_body(acc_ref, den_ref, h_ref, nt_ref, al_ref, wa_ref,
                    g_ref, b_ref, m_ref, o_ref):
    agg = acc_ref[0] + acc_ref[1]
    den = den_ref[0] + den_ref[1]
    div = jnp.dot(den, m_ref[...], preferred_element_type=jnp.float32)
    div = jnp.where(div > 0.0, div, 1.0)
    hagg = agg / div
    nt = nt_ref[...]
    hlin = jnp.zeros((hagg.shape[0], _D), jnp.float32)
    for t in range(_NT):
        hlin = hlin + jnp.where(
            nt == t, jnp.dot(hagg, wa_ref[t], preferred_element_type=jnp.float32), 0.0)
    al = al_ref[...]
    x = h_ref[...]
    hout = hlin * al + x * (1.0 - al)
    res = x + hout
    mu = jnp.mean(res, axis=-1, keepdims=True)
    var = jnp.mean((res - mu) ** 2, axis=-1, keepdims=True)
    o_ref[...] = (res - mu) / jnp.sqrt(var + 1e-5) * g_ref[...] + b_ref[...]


def _dense_out(acc2, den2, h, nt2, salpha, WaP, gamma2, beta2, M):
    return pl.pallas_call(
        _dense_out_body,
        grid=(_N // _BN,),
        in_specs=[pl.BlockSpec((_NC, _BN, _D), lambda i: (0, i, 0)),
                  pl.BlockSpec((_NC, _BN, _L), lambda i: (0, i, 0)),
                  pl.BlockSpec((_BN, _D), lambda i: (i, 0)),
                  pl.BlockSpec((_BN, 1), lambda i: (i, 0)),
                  pl.BlockSpec((_BN, 1), lambda i: (i, 0)),
                  pl.BlockSpec((_NT, _D, _D), lambda i: (0, 0, 0)),
                  pl.BlockSpec((1, _D), lambda i: (0, 0)),
                  pl.BlockSpec((1, _D), lambda i: (0, 0)),
                  pl.BlockSpec((_L, _D), lambda i: (0, 0))],
        out_specs=pl.BlockSpec((_BN, _D), lambda i: (i, 0)),
        out_shape=jax.ShapeDtypeStruct((_N, _D), jnp.float32),
    )(acc2, den2, h, nt2, salpha, WaP, gamma2, beta2, M)


def kernel(h, edge_index, ntype, etype, Wk, Wq, Wv, Wa, rel_pri, rel_att,
           rel_msg, skip, gamma, beta):
    J = jnp.asarray(_J)
    WqP = Wq[:, :, J]
    WaP = Wa[:, J, :]
    eyeH = jnp.eye(_H, dtype=jnp.float32)
    RattP = jnp.einsum('htio,hg->hitog', rel_att, eyeH).reshape(_D, _ET * _D)
    RmsgP = jnp.einsum('htio,hg->hitog', rel_msg, eyeH).reshape(_D, _ET * _D)
    privP = (jnp.broadcast_to(jnp.transpose(rel_pri)[:, None, :],
                              (_ET, _HS, _H)) / np.sqrt(_HS)).reshape(_ET * _D)
    RattPs = RattP * privP[None, :]
    nt2 = ntype[:, None]

    qP, krelP, vrelP = _dense_in(h, nt2, Wk, WqP, Wv, RattPs, RmsgP)
    krelT = krelP.reshape(_N * _ET, _D)
    vrelT = vrelP.reshape(_N * _ET, _D)

    acc2, denP = _edge_stage(krelT, vrelT, qP,
                             edge_index[0], edge_index[1], etype)
    den2 = denP.reshape(_NC, _NPAD, _L)

    salpha = jax.nn.sigmoid(skip)[ntype][:, None]
    return _dense_out(acc2, den2, h, nt2, salpha, WaP,
                      gamma[None, :], beta[None, :], jnp.asarray(_M_NP))


# speedup vs baseline: 49.7395x; 1.6129x over previous
import numpy as np
import jax
import jax.numpy as jnp
from jax import lax
from jax.experimental import pallas as pl
from jax.experimental.pallas import tpu as pltpu
from jax.experimental.pallas import tpu_sc as plsc

_N, _E, _D, _H, _HS, _NT, _ET = 10000, 320000, 128, 8, 16, 3, 5

_NC, _NS, _L = 2, 16, 16
_C = 40
_EPW = _E // (_NC * _NS)
_NCH = _EPW // _C
_NPAD = 10240
_RPT = _NPAD // _NS
_DN = _NPAD // 8
_DRPT = _DN // _NS
_BN = 1000

_J = np.array([(p % 8) * 16 + p // 8 for p in range(_D)], dtype=np.int32)
_M_NP = np.zeros((_L, _D), dtype=np.float32)
_M_NP[np.arange(_D) % _H, np.arange(_D)] = 1.0


def _dense_in_body(h_ref, nt_ref, wk_ref, wq_ref, wv_ref, ra_ref, rm_ref,
                   q_ref, kr_ref, vr_ref):
    x = h_ref[...]
    nt = nt_ref[...]

    def typed(w_ref):
        out = jnp.zeros((x.shape[0], _D), jnp.float32)
        for t in range(_NT):
            out = out + jnp.where(
                nt == t, jnp.dot(x, w_ref[t], preferred_element_type=jnp.float32), 0.0)
        return out

    k = typed(wk_ref)
    v = typed(wv_ref)
    q_ref[...] = typed(wq_ref)
    kr_ref[...] = jnp.dot(k, ra_ref[...], preferred_element_type=jnp.float32)
    vr_ref[...] = jnp.dot(v, rm_ref[...], preferred_element_type=jnp.float32)


def _dense_in(h, nt2, Wk, WqP, Wv, RattPs, RmsgP):
    full3 = pl.BlockSpec((_NT, _D, _D), lambda i: (0, 0, 0))
    rspec = pl.BlockSpec((_D, _ET * _D), lambda i: (0, 0))
    return pl.pallas_call(
        _dense_in_body,
        grid=(_N // _BN,),
        in_specs=[pl.BlockSpec((_BN, _D), lambda i: (i, 0)),
                  pl.BlockSpec((_BN, 1), lambda i: (i, 0)),
                  full3, full3, full3, rspec, rspec],
        out_specs=[pl.BlockSpec((_BN, _D), lambda i: (i, 0)),
                   pl.BlockSpec((_BN, _ET * _D), lambda i: (i, 0)),
                   pl.BlockSpec((_BN, _ET * _D), lambda i: (i, 0))],
        out_shape=[jax.ShapeDtypeStruct((_N, _D), jnp.float32),
                   jax.ShapeDtypeStruct((_N, _ET * _D), jnp.float32),
                   jax.ShapeDtypeStruct((_N, _ET * _D), jnp.float32)],
    )(h, nt2, Wk, WqP, Wv, RattPs, RmsgP)


def _edge_body(krelT, vrelT, qP, e3, accout, denout,
               acc_sh, den_sh, e1buf, dbuf, dpad, ibuf, nbuf,
               kbuf, qbuf, wbuf, debuf, sem0, sem1, sem2, sem3, sem4):
    cid = lax.axis_index("c")
    sid = lax.axis_index("s")
    iot = lax.iota(jnp.int32, _L)
    z16 = jnp.zeros((_L,), jnp.float32)
    zi16 = jnp.zeros((_L,), jnp.int32)

    def zrow(r, carry):
        for v2 in range(_D // _L):
            wbuf[r, pl.ds(v2 * _L, _L)] = z16
            debuf[r, pl.ds(v2 * _L, _L)] = z16
        return carry
    lax.fori_loop(0, _C, zrow, 0)
    for b in range(_RPT // _C):
        pltpu.sync_copy(wbuf, acc_sh.at[pl.ds(sid * _RPT + b * _C, _C)])
    for b in range(_DRPT // _C):
        pltpu.sync_copy(wbuf, den_sh.at[pl.ds(sid * _DRPT + b * _C, _C)])
    plsc.subcore_barrier()

    base = (cid * _NS + sid) * _EPW
    perm = iot ^ 8
    gdn = lax.GatherDimensionNumbers(
        offset_dims=(), collapsed_slice_dims=(0,), start_index_map=(0,))
    goffs = []
    o = 0
    while o + _L < _C:
        goffs.append(o)
        o += _L
    goffs.append(_C - _L)

    gbase = (cid * _NS + sid) * _NCH

    def chunk(j, carry):
        off = (gbase + j) * (3 * _C)
        pltpu.sync_copy(e3.at[pl.ds(off, 3 * _C)], e1buf)
        for go in goffs:
            sl = pl.ds(go, _L)
            dv = e1buf[pl.ds(_C + go, _L)]
            ibuf[sl] = e1buf[pl.ds(go, _L)] * _ET + e1buf[pl.ds(2 * _C + go, _L)]
            nbuf[sl] = lax.shift_right_logical(dv, 3)
            dbuf[sl] = dv
            dpad[sl] = dv
        cp0 = pltpu.async_copy(krelT.at[ibuf], kbuf, sem0)
        cp1 = pltpu.async_copy(qP.at[dbuf], qbuf, sem1)
        cp2 = pltpu.async_copy(vrelT.at[ibuf], wbuf, sem2)
        cp0.wait()
        cp1.wait()
        cp2.wait()

        def edge(c, ecarry):
            s16 = kbuf[c, pl.ds(0, _L)] * qbuf[c, pl.ds(0, _L)]
            for v2 in range(1, _D // _L):
                sl = pl.ds(v2 * _L, _L)
                s16 = s16 + kbuf[c, sl] * qbuf[c, sl]
            a16 = s16 + lax.gather(
                s16, perm[:, None], dimension_numbers=gdn, slice_sizes=(1,),
                mode=lax.GatherScatterMode.PROMISE_IN_BOUNDS)
            ex = jnp.exp(a16)
            slot = (dpad[pl.ds(c, _L)][0] & 7) * _L
            for v2 in range(_D // _L):
                sl = pl.ds(v2 * _L, _L)
                wbuf[c, sl] = wbuf[c, sl] * ex
                debuf[c, sl] = jnp.where(slot == v2 * _L, ex, z16)
            return ecarry
        lax.fori_loop(0, _C, edge, 0)

        pltpu.sync_copy(wbuf, acc_sh.at[dbuf], add=True)
        pltpu.sync_copy(debuf, den_sh.at[nbuf], add=True)
        return carry
    lax.fori_loop(0, _NCH, chunk, 0)

    plsc.subcore_barrier()
    pltpu.sync_copy(acc_sh.at[pl.ds(sid * _RPT, _RPT)],
                    accout.at[cid, pl.ds(sid * _RPT, _RPT)])
    pltpu.sync_copy(den_sh.at[pl.ds(sid * _DRPT, _DRPT)],
                    denout.at[cid, pl.ds(sid * _DRPT, _DRPT)])


def _edge_stage(krelT, vrelT, qP, e3):
    kern = pl.kernel(
        _edge_body,
        out_type=(jax.ShapeDtypeStruct((_NC, _NPAD, _D), jnp.float32),
                  jax.ShapeDtypeStruct((_NC, _DN, _D), jnp.float32)),
        mesh=plsc.VectorSubcoreMesh(core_axis_name="c", subcore_axis_name="s",
                                    num_cores=_NC, num_subcores=_NS),
        scratch_types=[
            pltpu.VMEM_SHARED((_NPAD, _D), jnp.float32),
            pltpu.VMEM_SHARED((_DN, _D), jnp.float32),
            pltpu.VMEM((3 * _C,), jnp.int32),
            pltpu.VMEM((_C,), jnp.int32),
            pltpu.VMEM((_C + _L,), jnp.int32),
            pltpu.VMEM((_C,), jnp.int32),
            pltpu.VMEM((_C,), jnp.int32),
            pltpu.VMEM((_C, _D), jnp.float32),
            pltpu.VMEM((_C, _D), jnp.float32),
            pltpu.VMEM((_C, _D), jnp.float32),
            pltpu.VMEM((_C, _D), jnp.float32),
            pltpu.SemaphoreType.DMA,
            pltpu.SemaphoreType.DMA,
            pltpu.SemaphoreType.DMA,
            pltpu.SemaphoreType.DMA,
            pltpu.SemaphoreType.DMA,
        ],
    )
    return kern(krelT, vrelT, qP, e3)


def _dense_out_body(acc_ref, den_ref, h_ref, nt_ref, al_ref, wa_ref,
                    g_ref, b_ref, m_ref, o_ref):
    agg = acc_ref[0] + acc_ref[1]
    den = den_ref[0] + den_ref[1]
    div = jnp.dot(den, m_ref[...], preferred_element_type=jnp.float32)
    div = jnp.where(div > 0.0, div, 1.0)
    hagg = agg / div
    nt = nt_ref[...]
    hlin = jnp.zeros((hagg.shape[0], _D), jnp.float32)
    for t in range(_NT):
        hlin = hlin + jnp.where(
            nt == t, jnp.dot(hagg, wa_ref[t], preferred_element_type=jnp.float32), 0.0)
    al = al_ref[...]
    x = h_ref[...]
    hout = hlin * al + x * (1.0 - al)
    res = x + hout
    mu = jnp.mean(res, axis=-1, keepdims=True)
    var = jnp.mean((res - mu) ** 2, axis=-1, keepdims=True)
    o_ref[...] = (res - mu) / jnp.sqrt(var + 1e-5) * g_ref[...] + b_ref[...]


def _dense_out(acc2, den2, h, nt2, salpha, WaP, gamma2, beta2, M):
    return pl.pallas_call(
        _dense_out_body,
        grid=(_N // _BN,),
        in_specs=[pl.BlockSpec((_NC, _BN, _D), lambda i: (0, i, 0)),
                  pl.BlockSpec((_NC, _BN, _L), lambda i: (0, i, 0)),
                  pl.BlockSpec((_BN, _D), lambda i: (i, 0)),
                  pl.BlockSpec((_BN, 1), lambda i: (i, 0)),
                  pl.BlockSpec((_BN, 1), lambda i: (i, 0)),
                  pl.BlockSpec((_NT, _D, _D), lambda i: (0, 0, 0)),
                  pl.BlockSpec((1, _D), lambda i: (0, 0)),
                  pl.BlockSpec((1, _D), lambda i: (0, 0)),
                  pl.BlockSpec((_L, _D), lambda i: (0, 0))],
        out_specs=pl.BlockSpec((_BN, _D), lambda i: (i, 0)),
        out_shape=jax.ShapeDtypeStruct((_N, _D), jnp.float32),
    )(acc2, den2, h, nt2, salpha, WaP, gamma2, beta2, M)


def kernel(h, edge_index, ntype, etype, Wk, Wq, Wv, Wa, rel_pri, rel_att,
           rel_msg, skip, gamma, beta):
    J = jnp.asarray(_J)
    WqP = Wq[:, :, J]
    WaP = Wa[:, J, :]
    eyeH = jnp.eye(_H, dtype=jnp.float32)
    RattP = jnp.einsum('htio,hg->hitog', rel_att, eyeH).reshape(_D, _ET * _D)
    RmsgP = jnp.einsum('htio,hg->hitog', rel_msg, eyeH).reshape(_D, _ET * _D)
    privP = (jnp.broadcast_to(jnp.transpose(rel_pri)[:, None, :],
                              (_ET, _HS, _H)) / np.sqrt(_HS)).reshape(_ET * _D)
    RattPs = RattP * privP[None, :]
    nt2 = ntype[:, None]

    qP, krelP, vrelP = _dense_in(h, nt2, Wk, WqP, Wv, RattPs, RmsgP)
    krelT = krelP.reshape(_N * _ET, _D)
    vrelT = vrelP.reshape(_N * _ET, _D)

    e3 = jnp.concatenate([edge_index, etype[None, :]], axis=0)
    e3c = jnp.transpose(e3.reshape(3, _E // _C, _C), (1, 0, 2)).reshape(-1)
    acc2, denP = _edge_stage(krelT, vrelT, qP, e3c)
    den2 = denP.reshape(_NC, _NPAD, _L)

    salpha = jax.nn.sigmoid(skip)[ntype][:, None]
    return _dense_out(acc2, den2, h, nt2, salpha, WaP,
                      gamma[None, :], beta[None, :], jnp.asarray(_M_NP))


# speedup vs baseline: 71.7070x; 1.4417x over previous
import numpy as np
import jax
import jax.numpy as jnp
from jax import lax
from jax.experimental import pallas as pl
from jax.experimental.pallas import tpu as pltpu
from jax.experimental.pallas import tpu_sc as plsc

_N, _E, _D, _H, _HS, _NT, _ET = 10000, 320000, 128, 8, 16, 3, 5

_NC, _NS, _L = 2, 16, 16
_C = 40
_EPW = _E // (_NC * _NS)
_NCH = _EPW // _C
_NPAD = 10240
_RPT = _NPAD // _NS
_DN = _NPAD // 8
_DRPT = _DN // _NS
_BN = 1000

_J = np.array([(p % 8) * 16 + p // 8 for p in range(_D)], dtype=np.int32)
_M_NP = np.zeros((_L, _D), dtype=np.float32)
_M_NP[np.arange(_D) % _H, np.arange(_D)] = 1.0


def _dense_in_body(h_ref, nt_ref, wk_ref, wq_ref, wv_ref, ra_ref, rm_ref,
                   q_ref, kr_ref, vr_ref):
    x = h_ref[...]
    nt = nt_ref[...]

    def typed(w_ref):
        out = jnp.zeros((x.shape[0], _D), jnp.float32)
        for t in range(_NT):
            out = out + jnp.where(
                nt == t, jnp.dot(x, w_ref[t], preferred_element_type=jnp.float32), 0.0)
        return out

    k = typed(wk_ref)
    v = typed(wv_ref)
    q_ref[...] = typed(wq_ref)
    kr_ref[...] = jnp.dot(k, ra_ref[...], preferred_element_type=jnp.float32)
    vr_ref[...] = jnp.dot(v, rm_ref[...], preferred_element_type=jnp.float32)


def _dense_in(h, nt2, Wk, WqP, Wv, RattPs, RmsgP):
    full3 = pl.BlockSpec((_NT, _D, _D), lambda i: (0, 0, 0))
    rspec = pl.BlockSpec((_D, _ET * _D), lambda i: (0, 0))
    return pl.pallas_call(
        _dense_in_body,
        grid=(_N // _BN,),
        in_specs=[pl.BlockSpec((_BN, _D), lambda i: (i, 0)),
                  pl.BlockSpec((_BN, 1), lambda i: (i, 0)),
                  full3, full3, full3, rspec, rspec],
        out_specs=[pl.BlockSpec((_BN, _D), lambda i: (i, 0)),
                   pl.BlockSpec((_BN, _ET * _D), lambda i: (i, 0)),
                   pl.BlockSpec((_BN, _ET * _D), lambda i: (i, 0))],
        out_shape=[jax.ShapeDtypeStruct((_N, _D), jnp.float32),
                   jax.ShapeDtypeStruct((_N, _ET * _D), jnp.float32),
                   jax.ShapeDtypeStruct((_N, _ET * _D), jnp.float32)],
    )(h, nt2, Wk, WqP, Wv, RattPs, RmsgP)


def _edge_body(krelT, vrelT, qP, e3, accout, denout,
               acc_sh, den_sh, e1buf, dbuf, dpad, ibuf, nbuf,
               kbuf, qbuf, wbuf, debuf, sem0, sem1, sem2, sem3, sem4):
    cid = lax.axis_index("c")
    sid = lax.axis_index("s")
    iot = lax.iota(jnp.int32, _L)
    z16 = jnp.zeros((_L,), jnp.float32)
    zi16 = jnp.zeros((_L,), jnp.int32)

    def zrow(r, carry):
        for v2 in range(_D // _L):
            wbuf[0, r, pl.ds(v2 * _L, _L)] = z16
            debuf[r, pl.ds(v2 * _L, _L)] = z16
        return carry
    lax.fori_loop(0, _C, zrow, 0)
    for b in range(_RPT // _C):
        pltpu.sync_copy(wbuf.at[0], acc_sh.at[pl.ds(sid * _RPT + b * _C, _C)])
    for b in range(_DRPT // _C):
        pltpu.sync_copy(wbuf.at[0], den_sh.at[pl.ds(sid * _DRPT + b * _C, _C)])
    plsc.subcore_barrier()

    base = (cid * _NS + sid) * _EPW
    perm = iot ^ 8
    gdn = lax.GatherDimensionNumbers(
        offset_dims=(), collapsed_slice_dims=(0,), start_index_map=(0,))
    goffs = []
    o = 0
    while o + _L < _C:
        goffs.append(o)
        o += _L
    goffs.append(_C - _L)

    gbase = (cid * _NS + sid) * _NCH

    def load_idx(j, sl2):
        off = (gbase + j) * (3 * _C)
        pltpu.sync_copy(e3.at[pl.ds(off, 3 * _C)], e1buf.at[sl2])
        for go in goffs:
            sl = pl.ds(go, _L)
            dv = e1buf[sl2, pl.ds(_C + go, _L)]
            ibuf[sl2, sl] = (e1buf[sl2, pl.ds(go, _L)] * _ET
                             + e1buf[sl2, pl.ds(2 * _C + go, _L)])
            nbuf[sl2, sl] = lax.shift_right_logical(dv, 3)
            dbuf[sl2, sl] = dv
            dpad[sl2, sl] = dv

    def fire(sl2):
        pltpu.async_copy(krelT.at[ibuf.at[sl2]], kbuf.at[sl2], sem0)
        pltpu.async_copy(qP.at[dbuf.at[sl2]], qbuf.at[sl2], sem1)
        pltpu.async_copy(vrelT.at[ibuf.at[sl2]], wbuf.at[sl2], sem2)

    def drain(sl2):
        pltpu.make_async_copy(krelT.at[ibuf.at[sl2]], kbuf.at[sl2], sem0).wait()
        pltpu.make_async_copy(qP.at[dbuf.at[sl2]], qbuf.at[sl2], sem1).wait()
        pltpu.make_async_copy(vrelT.at[ibuf.at[sl2]], wbuf.at[sl2], sem2).wait()

    load_idx(0, 0)
    fire(0)

    def pair(jp, carry):
        for s2 in range(2):
            j = jp * 2 + s2
            cur, nxt = s2, 1 - s2
            jn = jnp.minimum(j + 1, _NCH - 1)
            load_idx(jn, nxt)
            fire(nxt)
            drain(cur)

            def edge(c, ecarry):
                s16 = kbuf[cur, c, pl.ds(0, _L)] * qbuf[cur, c, pl.ds(0, _L)]
                for v2 in range(1, _D // _L):
                    sl = pl.ds(v2 * _L, _L)
                    s16 = s16 + kbuf[cur, c, sl] * qbuf[cur, c, sl]
                a16 = s16 + lax.gather(
                    s16, perm[:, None], dimension_numbers=gdn, slice_sizes=(1,),
                    mode=lax.GatherScatterMode.PROMISE_IN_BOUNDS)
                ex = jnp.exp(a16)
                slot = (dpad[cur, pl.ds(c, _L)][0] & 7) * _L
                for v2 in range(_D // _L):
                    sl = pl.ds(v2 * _L, _L)
                    wbuf[cur, c, sl] = wbuf[cur, c, sl] * ex
                    debuf[c, sl] = jnp.where(slot == v2 * _L, ex, z16)
                return ecarry
            lax.fori_loop(0, _C, edge, 0)

            pltpu.sync_copy(wbuf.at[cur], acc_sh.at[dbuf.at[cur]], add=True)
            pltpu.sync_copy(debuf, den_sh.at[nbuf.at[cur]], add=True)
        return carry
    lax.fori_loop(0, _NCH // 2, pair, 0)
    drain(0)

    plsc.subcore_barrier()
    pltpu.sync_copy(acc_sh.at[pl.ds(sid * _RPT, _RPT)],
                    accout.at[cid, pl.ds(sid * _RPT, _RPT)])
    pltpu.sync_copy(den_sh.at[pl.ds(sid * _DRPT, _DRPT)],
                    denout.at[cid, pl.ds(sid * _DRPT, _DRPT)])


def _edge_stage(krelT, vrelT, qP, e3):
    kern = pl.kernel(
        _edge_body,
        out_type=(jax.ShapeDtypeStruct((_NC, _NPAD, _D), jnp.float32),
                  jax.ShapeDtypeStruct((_NC, _DN, _D), jnp.float32)),
        mesh=plsc.VectorSubcoreMesh(core_axis_name="c", subcore_axis_name="s",
                                    num_cores=_NC, num_subcores=_NS),
        scratch_types=[
            pltpu.VMEM_SHARED((_NPAD, _D), jnp.float32),
            pltpu.VMEM_SHARED((_DN, _D), jnp.float32),
            pltpu.VMEM((2, 3 * _C), jnp.int32),
            pltpu.VMEM((2, _C), jnp.int32),
            pltpu.VMEM((2, _C + _L), jnp.int32),
            pltpu.VMEM((2, _C), jnp.int32),
            pltpu.VMEM((2, _C), jnp.int32),
            pltpu.VMEM((2, _C, _D), jnp.float32),
            pltpu.VMEM((2, _C, _D), jnp.float32),
            pltpu.VMEM((2, _C, _D), jnp.float32),
            pltpu.VMEM((_C, _D), jnp.float32),
            pltpu.SemaphoreType.DMA,
            pltpu.SemaphoreType.DMA,
            pltpu.SemaphoreType.DMA,
            pltpu.SemaphoreType.DMA,
            pltpu.SemaphoreType.DMA,
        ],
    )
    return kern(krelT, vrelT, qP, e3)


def _dense_out_body(acc_ref, den_ref, h_ref, nt_ref, al_ref, wa_ref,
                    g_ref, b_ref, m_ref, o_ref):
    agg = acc_ref[0] + acc_ref[1]
    den = den_ref[0] + den_ref[1]
    div = jnp.dot(den, m_ref[...], preferred_element_type=jnp.float32)
    div = jnp.where(div > 0.0, div, 1.0)
    hagg = agg / div
    nt = nt_ref[...]
    hlin = jnp.zeros((hagg.shape[0], _D), jnp.float32)
    for t in range(_NT):
        hlin = hlin + jnp.where(
            nt == t, jnp.dot(hagg, wa_ref[t], preferred_element_type=jnp.float32), 0.0)
    al = al_ref[...]
    x = h_ref[...]
    hout = hlin * al + x * (1.0 - al)
    res = x + hout
    mu = jnp.mean(res, axis=-1, keepdims=True)
    var = jnp.mean((res - mu) ** 2, axis=-1, keepdims=True)
    o_ref[...] = (res - mu) / jnp.sqrt(var + 1e-5) * g_ref[...] + b_ref[...]


def _dense_out(acc2, den2, h, nt2, salpha, WaP, gamma2, beta2, M):
    return pl.pallas_call(
        _dense_out_body,
        grid=(_N // _BN,),
        in_specs=[pl.BlockSpec((_NC, _BN, _D), lambda i: (0, i, 0)),
                  pl.BlockSpec((_NC, _BN, _L), lambda i: (0, i, 0)),
                  pl.BlockSpec((_BN, _D), lambda i: (i, 0)),
                  pl.BlockSpec((_BN, 1), lambda i: (i, 0)),
                  pl.BlockSpec((_BN, 1), lambda i: (i, 0)),
                  pl.BlockSpec((_NT, _D, _D), lambda i: (0, 0, 0)),
                  pl.BlockSpec((1, _D), lambda i: (0, 0)),
                  pl.BlockSpec((1, _D), lambda i: (0, 0)),
                  pl.BlockSpec((_L, _D), lambda i: (0, 0))],
        out_specs=pl.BlockSpec((_BN, _D), lambda i: (i, 0)),
        out_shape=jax.ShapeDtypeStruct((_N, _D), jnp.float32),
    )(acc2, den2, h, nt2, salpha, WaP, gamma2, beta2, M)


def kernel(h, edge_index, ntype, etype, Wk, Wq, Wv, Wa, rel_pri, rel_att,
           rel_msg, skip, gamma, beta):
    J = jnp.asarray(_J)
    WqP = Wq[:, :, J]
    WaP = Wa[:, J, :]
    eyeH = jnp.eye(_H, dtype=jnp.float32)
    RattP = jnp.einsum('htio,hg->hitog', rel_att, eyeH).reshape(_D, _ET * _D)
    RmsgP = jnp.einsum('htio,hg->hitog', rel_msg, eyeH).reshape(_D, _ET * _D)
    privP = (jnp.broadcast_to(jnp.transpose(rel_pri)[:, None, :],
                              (_ET, _HS, _H)) / np.sqrt(_HS)).reshape(_ET * _D)
    RattPs = RattP * privP[None, :]
    nt2 = ntype[:, None]

    qP, krelP, vrelP = _dense_in(h, nt2, Wk, WqP, Wv, RattPs, RmsgP)
    krelT = krelP.reshape(_N * _ET, _D)
    vrelT = vrelP.reshape(_N * _ET, _D)

    e3 = jnp.concatenate([edge_index, etype[None, :]], axis=0)
    e3c = jnp.transpose(e3.reshape(3, _E // _C, _C), (1, 0, 2)).reshape(-1)
    acc2, denP = _edge_stage(krelT, vrelT, qP, e3c)
    den2 = denP.reshape(_NC, _NPAD, _L)

    salpha = jax.nn.sigmoid(skip)[ntype][:, None]
    return _dense_out(acc2, den2, h, nt2, salpha, WaP,
                      gamma[None, :], beta[None, :], jnp.asarray(_M_NP))
